# baseline (speedup 1.0000x reference)
import jax
import jax.numpy as jnp
from jax.experimental import pallas as pl
from jax.experimental.pallas import tpu as pltpu

_NT = 1024
_KA = 16


def _split3(v):
    hi = v.astype(jnp.bfloat16)
    r = v - hi.astype(jnp.float32)
    mid = r.astype(jnp.bfloat16)
    lo = (r - mid.astype(jnp.float32)).astype(jnp.bfloat16)
    return hi, mid, lo


def _chamfer_tc_kernel(x_ref, yt_ref, row_tot_ref, col_tot_ref,
                       colmin_ref, xa_ref, ya_ref):
    b = pl.program_id(0)
    i = pl.program_id(1)
    ni = pl.num_programs(1)

    x = x_ref[0]
    yt = yt_ref[0]
    bf16 = jnp.bfloat16

    xsq = jnp.sum(x * x, axis=1, keepdims=True)
    xh, xm, xl = _split3(xsq)
    xa_ref[...] = jnp.zeros(xa_ref.shape, bf16)
    xa_ref[:, 0:3] = (-2.0 * x).astype(bf16)
    xa_ref[:, 3:4] = xh
    xa_ref[:, 4:5] = xm
    xa_ref[:, 5:6] = xl
    xa_ref[:, 6:9] = jnp.ones((x.shape[0], 3), bf16)

    @pl.when(i == 0)
    def _build_ya():
        ysq = jnp.sum(yt * yt, axis=0, keepdims=True)
        yh, ym, yl = _split3(ysq)
        ya_ref[...] = jnp.zeros(ya_ref.shape, bf16)
        ya_ref[0:3, :] = yt.astype(bf16)
        ya_ref[3:6, :] = jnp.ones((3, yt.shape[1]), bf16)
        ya_ref[6:7, :] = yh
        ya_ref[7:8, :] = ym
        ya_ref[8:9, :] = yl

    d = jax.lax.dot_general(
        xa_ref[...], ya_ref[...],
        dimension_numbers=(((1,), (0,)), ((), ())),
        preferred_element_type=jnp.float32,
    )

    row_min = jnp.min(d, axis=1, keepdims=True)
    col_min = jnp.min(d, axis=0, keepdims=True)

    @pl.when(jnp.logical_and(b == 0, i == 0))
    def _init():
        row_tot_ref[...] = jnp.zeros((1, 1), jnp.float32)
        col_tot_ref[...] = jnp.zeros((1, 1), jnp.float32)

    row_tot_ref[...] += jnp.sum(row_min, axis=0, keepdims=True)

    @pl.when(i == 0)
    def _colmin_init():
        colmin_ref[...] = col_min

    @pl.when(i > 0)
    def _colmin_acc():
        colmin_ref[...] = jnp.minimum(colmin_ref[...], col_min)

    @pl.when(i == ni - 1)
    def _colmin_finish():
        col_tot_ref[...] += jnp.sum(colmin_ref[...], axis=1, keepdims=True)


def kernel(pred, gt):
    B, N, D = pred.shape
    M = gt.shape[1]
    gt_t = jnp.swapaxes(gt, 1, 2)

    grid = (B, N // _NT)
    row_tot, col_tot = pl.pallas_call(
        _chamfer_tc_kernel,
        grid=grid,
        in_specs=[
            pl.BlockSpec((1, _NT, D), lambda b, i: (b, i, 0)),
            pl.BlockSpec((1, D, M), lambda b, i: (b, 0, 0)),
        ],
        out_specs=[
            pl.BlockSpec((1, 1), lambda b, i: (0, 0)),
            pl.BlockSpec((1, 1), lambda b, i: (0, 0)),
        ],
        out_shape=[
            jax.ShapeDtypeStruct((1, 1), jnp.float32),
            jax.ShapeDtypeStruct((1, 1), jnp.float32),
        ],
        scratch_shapes=[
            pltpu.VMEM((1, M), jnp.float32),
            pltpu.VMEM((_NT, _KA), jnp.bfloat16),
            pltpu.VMEM((_KA, M), jnp.bfloat16),
        ],
    )(pred, gt_t)

    return row_tot[0, 0] / (B * N) + col_tot[0, 0] / (B * M)


# speedup vs baseline: 1.0385x; 1.0385x over previous
import jax
import jax.numpy as jnp
from jax.experimental import pallas as pl
from jax.experimental.pallas import tpu as pltpu

_NT = 2048
_KA = 16


def _split3(v):
    hi = v.astype(jnp.bfloat16)
    r = v - hi.astype(jnp.float32)
    mid = r.astype(jnp.bfloat16)
    lo = (r - mid.astype(jnp.float32)).astype(jnp.bfloat16)
    return hi, mid, lo


def _chamfer_tc_kernel(x_ref, yt_ref, row_tot_ref, col_tot_ref,
                       colmin_ref, xa_ref, ya_ref):
    b = pl.program_id(0)
    i = pl.program_id(1)
    ni = pl.num_programs(1)

    x = x_ref[0]
    yt = yt_ref[0]
    bf16 = jnp.bfloat16

    xsq = jnp.sum(x * x, axis=1, keepdims=True)
    xh, xm, xl = _split3(xsq)
    xa_ref[...] = jnp.zeros(xa_ref.shape, bf16)
    xa_ref[:, 0:3] = (-2.0 * x).astype(bf16)
    xa_ref[:, 3:4] = xh
    xa_ref[:, 4:5] = xm
    xa_ref[:, 5:6] = xl
    xa_ref[:, 6:9] = jnp.ones((x.shape[0], 3), bf16)

    @pl.when(i == 0)
    def _build_ya():
        ysq = jnp.sum(yt * yt, axis=0, keepdims=True)
        yh, ym, yl = _split3(ysq)
        ya_ref[...] = jnp.zeros(ya_ref.shape, bf16)
        ya_ref[0:3, :] = yt.astype(bf16)
        ya_ref[3:6, :] = jnp.ones((3, yt.shape[1]), bf16)
        ya_ref[6:7, :] = yh
        ya_ref[7:8, :] = ym
        ya_ref[8:9, :] = yl

    d = jax.lax.dot_general(
        xa_ref[...], ya_ref[...],
        dimension_numbers=(((1,), (0,)), ((), ())),
        preferred_element_type=jnp.float32,
    )

    row_min = jnp.min(d, axis=1, keepdims=True)
    col_min = jnp.min(d, axis=0, keepdims=True)

    @pl.when(jnp.logical_and(b == 0, i == 0))
    def _init():
        row_tot_ref[...] = jnp.zeros((1, 1), jnp.float32)
        col_tot_ref[...] = jnp.zeros((1, 1), jnp.float32)

    row_tot_ref[...] += jnp.sum(row_min, axis=0, keepdims=True)

    @pl.when(i == 0)
    def _colmin_init():
        colmin_ref[...] = col_min

    @pl.when(i > 0)
    def _colmin_acc():
        colmin_ref[...] = jnp.minimum(colmin_ref[...], col_min)

    @pl.when(i == ni - 1)
    def _colmin_finish():
        col_tot_ref[...] += jnp.sum(colmin_ref[...], axis=1, keepdims=True)


def kernel(pred, gt):
    B, N, D = pred.shape
    M = gt.shape[1]
    gt_t = jnp.swapaxes(gt, 1, 2)

    grid = (B, N // _NT)
    row_tot, col_tot = pl.pallas_call(
        _chamfer_tc_kernel,
        grid=grid,
        in_specs=[
            pl.BlockSpec((1, _NT, D), lambda b, i: (b, i, 0)),
            pl.BlockSpec((1, D, M), lambda b, i: (b, 0, 0)),
        ],
        out_specs=[
            pl.BlockSpec((1, 1), lambda b, i: (0, 0)),
            pl.BlockSpec((1, 1), lambda b, i: (0, 0)),
        ],
        out_shape=[
            jax.ShapeDtypeStruct((1, 1), jnp.float32),
            jax.ShapeDtypeStruct((1, 1), jnp.float32),
        ],
        scratch_shapes=[
            pltpu.VMEM((1, M), jnp.float32),
            pltpu.VMEM((_NT, _KA), jnp.bfloat16),
            pltpu.VMEM((_KA, M), jnp.bfloat16),
        ],
    )(pred, gt_t)

    return row_tot[0, 0] / (B * N) + col_tot[0, 0] / (B * M)


# speedup vs baseline: 1.0808x; 1.0407x over previous
import jax
import jax.numpy as jnp
from jax.experimental import pallas as pl
from jax.experimental.pallas import tpu as pltpu

_NT = 2048
_MT = 1024
_KA = 16


def _split3(v):
    hi = v.astype(jnp.bfloat16)
    r = v - hi.astype(jnp.float32)
    mid = r.astype(jnp.bfloat16)
    lo = (r - mid.astype(jnp.float32)).astype(jnp.bfloat16)
    return hi, mid, lo


def _chamfer_tc_kernel(x_ref, yt_ref, row_tot_ref, col_tot_ref,
                       colmin_ref, xa_ref, ya_ref):
    b = pl.program_id(0)
    i = pl.program_id(1)
    ni = pl.num_programs(1)

    x = x_ref[0]
    yt = yt_ref[0]
    bf16 = jnp.bfloat16

    xsq = jnp.sum(x * x, axis=1, keepdims=True)
    xh, xm, xl = _split3(xsq)
    xa_ref[...] = jnp.zeros(xa_ref.shape, bf16)
    xa_ref[:, 0:3] = (-2.0 * x).astype(bf16)
    xa_ref[:, 3:4] = xh
    xa_ref[:, 4:5] = xm
    xa_ref[:, 5:6] = xl
    xa_ref[:, 6:9] = jnp.ones((x.shape[0], 3), bf16)

    @pl.when(i == 0)
    def _build_ya():
        ysq = jnp.sum(yt * yt, axis=0, keepdims=True)
        yh, ym, yl = _split3(ysq)
        ya_ref[...] = jnp.zeros(ya_ref.shape, bf16)
        ya_ref[0:3, :] = yt.astype(bf16)
        ya_ref[3:6, :] = jnp.ones((3, yt.shape[1]), bf16)
        ya_ref[6:7, :] = yh
        ya_ref[7:8, :] = ym
        ya_ref[8:9, :] = yl

    @pl.when(jnp.logical_and(b == 0, i == 0))
    def _init():
        row_tot_ref[...] = jnp.zeros((1, 1), jnp.float32)
        col_tot_ref[...] = jnp.zeros((1, 1), jnp.float32)

    @pl.when(i == 0)
    def _colmin_init():
        colmin_ref[...] = jnp.full(colmin_ref.shape, jnp.inf, jnp.float32)

    M = ya_ref.shape[1]
    xa = xa_ref[...]
    row_mins = []
    for j in range(M // _MT):
        sl = pl.ds(j * _MT, _MT)
        dj = jax.lax.dot_general(
            xa, ya_ref[:, sl],
            dimension_numbers=(((1,), (0,)), ((), ())),
            preferred_element_type=jnp.float32,
        )
        row_mins.append(jnp.min(dj, axis=1, keepdims=True))
        colmin_ref[0:1, sl] = jnp.minimum(
            colmin_ref[0:1, sl], jnp.min(dj, axis=0, keepdims=True))

    while len(row_mins) > 1:
        row_mins = [jnp.minimum(a, b) for a, b in zip(row_mins[::2], row_mins[1::2])]
    row_min = row_mins[0]

    row_tot_ref[...] += jnp.sum(row_min, axis=0, keepdims=True)

    @pl.when(i == ni - 1)
    def _colmin_finish():
        col_tot_ref[...] += jnp.sum(colmin_ref[...], axis=1, keepdims=True)


def kernel(pred, gt):
    B, N, D = pred.shape
    M = gt.shape[1]
    gt_t = jnp.swapaxes(gt, 1, 2)

    grid = (B, N // _NT)
    row_tot, col_tot = pl.pallas_call(
        _chamfer_tc_kernel,
        grid=grid,
        in_specs=[
            pl.BlockSpec((1, _NT, D), lambda b, i: (b, i, 0)),
            pl.BlockSpec((1, D, M), lambda b, i: (b, 0, 0)),
        ],
        out_specs=[
            pl.BlockSpec((1, 1), lambda b, i: (0, 0)),
            pl.BlockSpec((1, 1), lambda b, i: (0, 0)),
        ],
        out_shape=[
            jax.ShapeDtypeStruct((1, 1), jnp.float32),
            jax.ShapeDtypeStruct((1, 1), jnp.float32),
        ],
        scratch_shapes=[
            pltpu.VMEM((1, M), jnp.float32),
            pltpu.VMEM((_NT, _KA), jnp.bfloat16),
            pltpu.VMEM((_KA, M), jnp.bfloat16),
        ],
    )(pred, gt_t)

    return row_tot[0, 0] / (B * N) + col_tot[0, 0] / (B * M)


# speedup vs baseline: 1.1036x; 1.0211x over previous
import jax
import jax.numpy as jnp
from jax.experimental import pallas as pl
from jax.experimental.pallas import tpu as pltpu

_MT = 1024
_KA = 16


def _split3(v):
    hi = v.astype(jnp.bfloat16)
    r = v - hi.astype(jnp.float32)
    mid = r.astype(jnp.bfloat16)
    lo = (r - mid.astype(jnp.float32)).astype(jnp.bfloat16)
    return hi, mid, lo


def _chamfer_tc_kernel(x_ref, yt_ref, row_tot_ref, col_tot_ref,
                       xa_ref, ya_ref):
    b = pl.program_id(0)

    x = x_ref[0]
    yt = yt_ref[0]
    bf16 = jnp.bfloat16
    N = x.shape[0]
    M = yt.shape[1]

    xsq = jnp.sum(x * x, axis=1, keepdims=True)
    xh, xm, xl = _split3(xsq)
    xa_ref[...] = jnp.zeros(xa_ref.shape, bf16)
    xa_ref[:, 0:3] = (-2.0 * x).astype(bf16)
    xa_ref[:, 3:4] = xh
    xa_ref[:, 4:5] = xm
    xa_ref[:, 5:6] = xl
    xa_ref[:, 6:9] = jnp.ones((N, 3), bf16)

    ysq = jnp.sum(yt * yt, axis=0, keepdims=True)
    yh, ym, yl = _split3(ysq)
    ya_ref[...] = jnp.zeros(ya_ref.shape, bf16)
    ya_ref[0:3, :] = yt.astype(bf16)
    ya_ref[3:6, :] = jnp.ones((3, M), bf16)
    ya_ref[6:7, :] = yh
    ya_ref[7:8, :] = ym
    ya_ref[8:9, :] = yl

    @pl.when(b == 0)
    def _init():
        row_tot_ref[...] = jnp.zeros((1, 1), jnp.float32)
        col_tot_ref[...] = jnp.zeros((1, 1), jnp.float32)

    xa = xa_ref[...]
    row_mins = []
    col_sums = []
    for j in range(M // _MT):
        sl = pl.ds(j * _MT, _MT)
        dj = jax.lax.dot_general(
            xa, ya_ref[:, sl],
            dimension_numbers=(((1,), (0,)), ((), ())),
            preferred_element_type=jnp.float32,
        )
        row_mins.append(jnp.min(dj, axis=1, keepdims=True))
        cm = jnp.min(dj, axis=0, keepdims=True)
        col_sums.append(jnp.sum(cm, axis=1, keepdims=True))

    while len(row_mins) > 1:
        row_mins = [jnp.minimum(p, q)
                    for p, q in zip(row_mins[::2], row_mins[1::2])]
    row_min = row_mins[0]

    row_tot_ref[...] += jnp.sum(row_min, axis=0, keepdims=True)
    col_tot_ref[...] += sum(col_sums)


def kernel(pred, gt):
    B, N, D = pred.shape
    M = gt.shape[1]
    gt_t = jnp.swapaxes(gt, 1, 2)

    row_tot, col_tot = pl.pallas_call(
        _chamfer_tc_kernel,
        grid=(B,),
        in_specs=[
            pl.BlockSpec((1, N, D), lambda b: (b, 0, 0)),
            pl.BlockSpec((1, D, M), lambda b: (b, 0, 0)),
        ],
        out_specs=[
            pl.BlockSpec((1, 1), lambda b: (0, 0)),
            pl.BlockSpec((1, 1), lambda b: (0, 0)),
        ],
        out_shape=[
            jax.ShapeDtypeStruct((1, 1), jnp.float32),
            jax.ShapeDtypeStruct((1, 1), jnp.float32),
        ],
        scratch_shapes=[
            pltpu.VMEM((N, _KA), jnp.bfloat16),
            pltpu.VMEM((_KA, M), jnp.bfloat16),
        ],
    )(pred, gt_t)

    return row_tot[0, 0] / (B * N) + col_tot[0, 0] / (B * M)
